# baseline (speedup 1.0000x reference)
import functools

import jax
import jax.numpy as jnp
import numpy as np
from jax import lax
from jax.experimental import pallas as pl
from jax.experimental.pallas import tpu as pltpu
from jax.experimental.pallas import tpu_sc as plsc

_N = 10000
_E = 320000
_D = 128

_NCORES = 2
_NSUB = 16
_NTILES = _NCORES * _NSUB
_LANES = 128
_CH = -(-_E // (_NTILES * _LANES))
_CH = -(-_CH // 16) * 16
_CHH = _CH // 2
_EPAD = _NTILES * _CH * _LANES
_NP = 10240
_RPT = _NP // _NSUB
_mesh = plsc.VectorSubcoreMesh(core_axis_name="c", subcore_axis_name="s")

_BR = 1000
_GRID = _N // _BR
_HR = 128
_HPT = _HR // _NSUB


@functools.partial(
    pl.kernel,
    out_type=jax.ShapeDtypeStruct((_NCORES, _HR, _D), jnp.float32),
    mesh=_mesh,
    scratch_types=[
        pltpu.VMEM((_CH, _LANES), jnp.int32),
        pltpu.VMEM((_HR, _D), jnp.float32),
        pltpu.VMEM((1, _HR), jnp.int32),
        pltpu.VMEM_SHARED((_HR, _D), jnp.float32),
    ],
    compiler_params=pltpu.CompilerParams(needs_layout_passes=False),
)
def _deg_kernel(dsts, zeros, iota, out, dst_v, hist, idxh, acc):
    c = lax.axis_index("c")
    s = lax.axis_index("s")
    g = c * _NSUB + s
    pltpu.sync_copy(dsts.at[g], dst_v)
    pltpu.sync_copy(iota, idxh)
    pltpu.sync_copy(zeros.at[pl.ds(0, _HR)], hist)
    pltpu.sync_copy(zeros.at[pl.ds(s * _HPT, _HPT)], acc.at[pl.ds(s * _HPT, _HPT)])

    @pl.loop(0, _CH)
    def _(j):
        for k in range(8):
            idx = dst_v[j, pl.ds(k * 16, 16)]
            blk = (idx * 67109) >> 26
            fpos = blk * 1024 + (idx - blk * _BR)
            row = fpos >> 7
            col = fpos & 127
            cnt, last = plsc.scan_count(idx)
            plsc.addupdate_scatter(hist, [row, col],
                                   cnt.astype(jnp.float32), mask=last)

    plsc.subcore_barrier()
    pltpu.sync_copy(hist, acc.at[idxh.at[0]], add=True)
    plsc.subcore_barrier()
    pltpu.sync_copy(acc.at[pl.ds(s * _HPT, _HPT)], out.at[c, pl.ds(s * _HPT, _HPT)])


@functools.partial(
    pl.kernel,
    out_type=jax.ShapeDtypeStruct((_NCORES, _NP, _D), jnp.float32),
    mesh=_mesh,
    scratch_types=[
        pltpu.VMEM((_CHH, _LANES), jnp.int32),
        pltpu.VMEM((_CHH, _LANES), jnp.int32),
        pltpu.VMEM((_LANES, _D), jnp.float32),
        pltpu.VMEM((_LANES, _D), jnp.float32),
        pltpu.VMEM_SHARED((_NP, _D), jnp.float32),
        pltpu.SemaphoreType.DMA,
        pltpu.SemaphoreType.DMA,
        pltpu.SemaphoreType.DMA,
        pltpu.SemaphoreType.DMA,
    ],
)
def _agg_kernel(h, srcs, dsts, zeros, out, src_v, dst_v, rows_a, rows_b, acc,
                gsem_a, gsem_b, ssem_a, ssem_b):
    c = lax.axis_index("c")
    s = lax.axis_index("s")
    g = c * _NSUB + s
    r0 = s * _RPT
    pltpu.sync_copy(zeros.at[pl.ds(r0, _RPT)], acc.at[pl.ds(r0, _RPT)])
    plsc.subcore_barrier()

    for half in range(2):
        pltpu.sync_copy(srcs.at[g, pl.ds(half * _CHH, _CHH)], src_v)
        pltpu.sync_copy(dsts.at[g, pl.ds(half * _CHH, _CHH)], dst_v)
        pltpu.async_copy(h.at[src_v.at[0]], rows_a, gsem_a)
        pltpu.async_copy(h.at[src_v.at[1]], rows_b, gsem_b)

        @pl.loop(0, _CHH, step=2)
        def _(j):
            pltpu.make_async_copy(h.at[src_v.at[j]], rows_a, gsem_a).wait()
            pltpu.sync_copy(rows_a, acc.at[dst_v.at[j]], add=True)

            @pl.when(j + 2 < _CHH)
            def _():
                pltpu.async_copy(h.at[src_v.at[j + 2]], rows_a, gsem_a)

            pltpu.make_async_copy(h.at[src_v.at[j + 1]], rows_b, gsem_b).wait()
            pltpu.sync_copy(rows_b, acc.at[dst_v.at[j + 1]], add=True)

            @pl.when(j + 3 < _CHH)
            def _():
                pltpu.async_copy(h.at[src_v.at[j + 3]], rows_b, gsem_b)

    plsc.subcore_barrier()
    pltpu.sync_copy(acc.at[pl.ds(r0, _RPT)], out.at[c, pl.ds(r0, _RPT)])


_EXA = np.zeros((_BR, 8), np.float32)
_EXA[np.arange(_BR), np.arange(_BR) >> 7] = 1.0
_EXM = np.zeros((_BR, _D), np.float32)
_EXM[np.arange(_BR), np.arange(_BR) & 127] = 1.0


def _dinv(p_ref, a_ref, m_ref):
    degp = p_ref[0] + p_ref[1] + 1.0
    dp = lax.rsqrt(degp)
    e = jnp.dot(a_ref[...], dp, preferred_element_type=jnp.float32,
                precision=lax.Precision.HIGHEST)
    return jnp.sum(e * m_ref[...], axis=1, keepdims=True)


def _l1_body(p_ref, a_ref, m_ref, x_ref, w_ref, o_ref):
    o_ref[...] = jnp.dot(x_ref[...] * _dinv(p_ref, a_ref, m_ref), w_ref[...],
                         preferred_element_type=jnp.float32)


def _l2_body(p_ref, a_ref, m_ref, s_ref, hp_ref, b_ref, w_ref, o_ref):
    dinv = _dinv(p_ref, a_ref, m_ref)
    tot = s_ref[0] + s_ref[1] + hp_ref[...]
    act = jnp.maximum(tot * dinv + b_ref[...], 0.0)
    o_ref[...] = jnp.dot(act * dinv, w_ref[...],
                         preferred_element_type=jnp.float32)


def _out_body(p_ref, a_ref, m_ref, s_ref, hp_ref, b_ref, o_ref):
    dinv = _dinv(p_ref, a_ref, m_ref)
    tot = s_ref[0] + s_ref[1] + hp_ref[...]
    o_ref[...] = jnp.maximum(tot * dinv + b_ref[...], 0.0)


_p_spec = pl.BlockSpec((_NCORES, 8, _D), lambda i: (0, i, 0))
_a_spec = pl.BlockSpec((_BR, 8), lambda i: (0, 0))
_m_spec = pl.BlockSpec((_BR, _D), lambda i: (0, 0))
_pair_spec = pl.BlockSpec((_NCORES, _BR, _D), lambda i: (0, i, 0))
_row_spec = pl.BlockSpec((_BR, _D), lambda i: (i, 0))
_w_spec = pl.BlockSpec((_D, _D), lambda i: (0, 0))
_b_spec = pl.BlockSpec((1, _D), lambda i: (0, 0))
_rows_out = jax.ShapeDtypeStruct((_N, _D), jnp.float32)

_l1 = pl.pallas_call(
    _l1_body, grid=(_GRID,),
    in_specs=[_p_spec, _a_spec, _m_spec, _row_spec, _w_spec],
    out_specs=_row_spec, out_shape=_rows_out)

_l2 = pl.pallas_call(
    _l2_body, grid=(_GRID,),
    in_specs=[_p_spec, _a_spec, _m_spec, _pair_spec, _row_spec, _b_spec, _w_spec],
    out_specs=_row_spec, out_shape=_rows_out)

_out_tc = pl.pallas_call(
    _out_body, grid=(_GRID,),
    in_specs=[_p_spec, _a_spec, _m_spec, _pair_spec, _row_spec, _b_spec],
    out_specs=_row_spec, out_shape=_rows_out)


def kernel(x, edge_index, W1, b1, W2, b2):
    n, d = x.shape
    e = edge_index.shape[1]
    pad = _EPAD - e
    padi = jnp.arange(pad, dtype=jnp.int32)
    srcs = jnp.concatenate(
        [edge_index[0], padi % n]).reshape(_NTILES, _CH, _LANES)
    dsts = jnp.concatenate(
        [edge_index[1], n + padi % (_NP - n)]).reshape(_NTILES, _CH, _LANES)
    zeros = jnp.zeros((_NP, _D), jnp.float32)
    exa = jnp.asarray(_EXA)
    exm = jnp.asarray(_EXM)
    b1r = b1.reshape(1, _D)
    b2r = b2.reshape(1, _D)

    p = _deg_kernel(dsts, zeros, jnp.arange(_HR, dtype=jnp.int32).reshape(1, _HR))
    h1 = _l1(p, exa, exm, x, W1)
    s1 = _agg_kernel(h1, srcs, dsts, zeros)
    h2 = _l2(p, exa, exm, s1, h1, b1r, W2)
    s2 = _agg_kernel(h2, srcs, dsts, zeros)
    return _out_tc(p, exa, exm, s2, h2, b2r)


# speedup vs baseline: 1.0399x; 1.0399x over previous
import functools

import jax
import jax.numpy as jnp
import numpy as np
from jax import lax
from jax.experimental import pallas as pl
from jax.experimental.pallas import tpu as pltpu
from jax.experimental.pallas import tpu_sc as plsc

_N = 10000
_E = 320000
_D = 128

_NCORES = 2
_NSUB = 16
_NTILES = _NCORES * _NSUB
_LANES = 128
_CH = -(-_E // (_NTILES * _LANES))
_CH = -(-_CH // 16) * 16
_CHH = _CH // 2
_EPAD = _NTILES * _CH * _LANES
_NP = 10240
_RPT = _NP // _NSUB
_mesh = plsc.VectorSubcoreMesh(core_axis_name="c", subcore_axis_name="s")

_BR = 1000
_GRID = _N // _BR
_HR = 128
_HPT = _HR // _NSUB


@functools.partial(
    pl.kernel,
    out_type=jax.ShapeDtypeStruct((_NCORES, _HR, _D), jnp.float32),
    mesh=_mesh,
    scratch_types=[
        pltpu.VMEM((_CH, _LANES), jnp.int32),
        pltpu.VMEM((_HR, _D), jnp.float32),
        pltpu.VMEM((1, _HR), jnp.int32),
        pltpu.VMEM_SHARED((_HR, _D), jnp.float32),
    ],
    compiler_params=pltpu.CompilerParams(needs_layout_passes=False),
)
def _deg_kernel(dsts, zeros, iota, out, dst_v, hist, idxh, acc):
    c = lax.axis_index("c")
    s = lax.axis_index("s")
    g = c * _NSUB + s
    pltpu.sync_copy(dsts.at[g], dst_v)
    pltpu.sync_copy(iota, idxh)
    pltpu.sync_copy(zeros.at[pl.ds(0, _HR)], hist)
    pltpu.sync_copy(zeros.at[pl.ds(s * _HPT, _HPT)], acc.at[pl.ds(s * _HPT, _HPT)])

    @pl.loop(0, _CH)
    def _(j):
        for k in range(8):
            idx = dst_v[j, pl.ds(k * 16, 16)]
            blk = (idx * 67109) >> 26
            fpos = blk * 1024 + (idx - blk * _BR)
            row = fpos >> 7
            col = fpos & 127
            cnt, last = plsc.scan_count(idx)
            plsc.addupdate_scatter(hist, [row, col],
                                   cnt.astype(jnp.float32), mask=last)

    plsc.subcore_barrier()
    pltpu.sync_copy(hist, acc.at[idxh.at[0]], add=True)
    plsc.subcore_barrier()
    pltpu.sync_copy(acc.at[pl.ds(s * _HPT, _HPT)], out.at[c, pl.ds(s * _HPT, _HPT)])


@functools.partial(
    pl.kernel,
    out_type=jax.ShapeDtypeStruct((_NCORES, _NP, _D), jnp.float32),
    mesh=_mesh,
    scratch_types=[
        pltpu.VMEM((_CHH, _LANES), jnp.int32),
        pltpu.VMEM((_CHH, _LANES), jnp.int32),
        pltpu.VMEM((_LANES, _D), jnp.float32),
        pltpu.VMEM((_LANES, _D), jnp.float32),
        pltpu.VMEM_SHARED((_NP, _D), jnp.float32),
        pltpu.SemaphoreType.DMA,
        pltpu.SemaphoreType.DMA,
        pltpu.SemaphoreType.DMA,
        pltpu.SemaphoreType.DMA,
    ],
)
def _agg_kernel(h, srcs, dsts, zeros, out, src_v, dst_v, rows_a, rows_b, acc,
                gsem_a, gsem_b, ssem_a, ssem_b):
    c = lax.axis_index("c")
    s = lax.axis_index("s")
    g = c * _NSUB + s
    r0 = s * _RPT
    pltpu.sync_copy(zeros.at[pl.ds(r0, _RPT)], acc.at[pl.ds(r0, _RPT)])
    plsc.subcore_barrier()

    for half in range(2):
        pltpu.sync_copy(srcs.at[g, pl.ds(half * _CHH, _CHH)], src_v)
        pltpu.sync_copy(dsts.at[g, pl.ds(half * _CHH, _CHH)], dst_v)
        pltpu.async_copy(h.at[src_v.at[0]], rows_a, gsem_a)
        pltpu.async_copy(h.at[src_v.at[1]], rows_b, gsem_b)

        @pl.loop(0, _CHH, step=2)
        def _(j):
            pltpu.make_async_copy(h.at[src_v.at[j]], rows_a, gsem_a).wait()
            pltpu.sync_copy(rows_a, acc.at[dst_v.at[j]], add=True)

            @pl.when(j + 2 < _CHH)
            def _():
                pltpu.async_copy(h.at[src_v.at[j + 2]], rows_a, gsem_a)

            pltpu.make_async_copy(h.at[src_v.at[j + 1]], rows_b, gsem_b).wait()
            pltpu.sync_copy(rows_b, acc.at[dst_v.at[j + 1]], add=True)

            @pl.when(j + 3 < _CHH)
            def _():
                pltpu.async_copy(h.at[src_v.at[j + 3]], rows_b, gsem_b)

    plsc.subcore_barrier()
    pltpu.sync_copy(acc.at[pl.ds(r0, _RPT)], out.at[c, pl.ds(r0, _RPT)])


_EXA = np.zeros((_BR, 8), np.float32)
_EXA[np.arange(_BR), np.arange(_BR) >> 7] = 1.0
_EXM = np.zeros((_BR, _D), np.float32)
_EXM[np.arange(_BR), np.arange(_BR) & 127] = 1.0


def _dinv(p_ref, a_ref, m_ref):
    degp = p_ref[0] + p_ref[1]
    e = jnp.dot(a_ref[...], degp, preferred_element_type=jnp.float32)
    deg = jnp.sum(e * m_ref[...], axis=1, keepdims=True) + 1.0
    return lax.rsqrt(deg)


def _l1_body(p_ref, a_ref, m_ref, x_ref, w_ref, o_ref):
    o_ref[...] = jnp.dot(x_ref[...] * _dinv(p_ref, a_ref, m_ref), w_ref[...],
                         preferred_element_type=jnp.float32)


def _l2_body(p_ref, a_ref, m_ref, s_ref, hp_ref, b_ref, w_ref, o_ref):
    dinv = _dinv(p_ref, a_ref, m_ref)
    tot = s_ref[0] + s_ref[1] + hp_ref[...]
    act = jnp.maximum(tot * dinv + b_ref[...], 0.0)
    o_ref[...] = jnp.dot(act * dinv, w_ref[...],
                         preferred_element_type=jnp.float32)


def _out_body(p_ref, a_ref, m_ref, s_ref, hp_ref, b_ref, o_ref):
    dinv = _dinv(p_ref, a_ref, m_ref)
    tot = s_ref[0] + s_ref[1] + hp_ref[...]
    o_ref[...] = jnp.maximum(tot * dinv + b_ref[...], 0.0)


_p_spec = pl.BlockSpec((_NCORES, 8, _D), lambda i: (0, i, 0))
_a_spec = pl.BlockSpec((_BR, 8), lambda i: (0, 0))
_m_spec = pl.BlockSpec((_BR, _D), lambda i: (0, 0))
_pair_spec = pl.BlockSpec((_NCORES, _BR, _D), lambda i: (0, i, 0))
_row_spec = pl.BlockSpec((_BR, _D), lambda i: (i, 0))
_w_spec = pl.BlockSpec((_D, _D), lambda i: (0, 0))
_b_spec = pl.BlockSpec((1, _D), lambda i: (0, 0))
_rows_out = jax.ShapeDtypeStruct((_N, _D), jnp.float32)

_l1 = pl.pallas_call(
    _l1_body, grid=(_GRID,),
    in_specs=[_p_spec, _a_spec, _m_spec, _row_spec, _w_spec],
    out_specs=_row_spec, out_shape=_rows_out)

_l2 = pl.pallas_call(
    _l2_body, grid=(_GRID,),
    in_specs=[_p_spec, _a_spec, _m_spec, _pair_spec, _row_spec, _b_spec, _w_spec],
    out_specs=_row_spec, out_shape=_rows_out)

_out_tc = pl.pallas_call(
    _out_body, grid=(_GRID,),
    in_specs=[_p_spec, _a_spec, _m_spec, _pair_spec, _row_spec, _b_spec],
    out_specs=_row_spec, out_shape=_rows_out)


def kernel(x, edge_index, W1, b1, W2, b2):
    n, d = x.shape
    e = edge_index.shape[1]
    pad = _EPAD - e
    padi = jnp.arange(pad, dtype=jnp.int32)
    srcs = jnp.concatenate(
        [edge_index[0], padi % n]).reshape(_NTILES, _CH, _LANES)
    dsts = jnp.concatenate(
        [edge_index[1], n + padi % (_NP - n)]).reshape(_NTILES, _CH, _LANES)
    zeros = jnp.zeros((_NP, _D), jnp.float32)
    exa = jnp.asarray(_EXA)
    exm = jnp.asarray(_EXM)
    b1r = b1.reshape(1, _D)
    b2r = b2.reshape(1, _D)

    p = _deg_kernel(dsts, zeros, jnp.arange(_HR, dtype=jnp.int32).reshape(1, _HR))
    h1 = _l1(p, exa, exm, x, W1)
    s1 = _agg_kernel(h1, srcs, dsts, zeros)
    h2 = _l2(p, exa, exm, s1, h1, b1r, W2)
    s2 = _agg_kernel(h2, srcs, dsts, zeros)
    return _out_tc(p, exa, exm, s2, h2, b2r)
